# baseline (speedup 1.0000x reference)
import functools

import jax
import jax.numpy as jnp
from jax import lax
from jax.experimental import pallas as pl
from jax.experimental.pallas import tpu as pltpu
from jax.experimental.pallas import tpu_sc as plsc

VOCAB = 1000000
H = 64
PAD = 0
SCALE = float(H) ** 0.5

B = 4096 * 200
NC, NS, L = 2, 16, 16
NW = NC * NS
B_PER_W = B // NW
CHUNK = 800
N_CHUNKS = B_PER_W // CHUNK
GRP = CHUNK // L


def _emb_body(idx_hbm, table_hbm, out_hbm,
              idx0, idx1, rows0, rows1,
              s_i0, s_i1, s_g0, s_g1, s_o0, s_o1):
    wid = lax.axis_index("c") * NS + lax.axis_index("s")
    base0 = wid * B_PER_W
    idx_b, rows_b = (idx0, idx1), (rows0, rows1)
    s_i, s_g, s_o = (s_i0, s_i1), (s_g0, s_g1), (s_o0, s_o1)

    def idx_copy(t, b):
        return pltpu.make_async_copy(
            idx_hbm.at[pl.ds(base0 + t * CHUNK, CHUNK)], idx_b[b], s_i[b])

    def gather_copy(b):
        return pltpu.make_async_copy(table_hbm.at[idx_b[b]], rows_b[b], s_g[b])

    def out_copy(t, b):
        return pltpu.make_async_copy(
            rows_b[b],
            out_hbm.at[pl.ds(base0 + t * CHUNK, CHUNK), pl.ds(0, H)],
            s_o[b])

    def compute(b):
        idx_v, rows_v = idx_b[b], rows_b[b]

        def grp_body(g, c2):
            row0 = g * L
            idxv = idx_v[pl.ds(row0, L)]
            pad_m = idxv == PAD
            npad = jnp.sum(jnp.where(pad_m, 1, 0).astype(jnp.int32))

            @pl.when(npad > 0)
            def _():
                rowids = row0 + lax.iota(jnp.int32, L)
                zeros = jnp.zeros((L,), jnp.float32)
                for c in range(H):
                    colids = jnp.full((L,), c, jnp.int32)
                    plsc.store_scatter(rows_v, [rowids, colids], zeros,
                                       mask=pad_m)
            return c2

        lax.fori_loop(0, GRP, grp_body, 0)

    idx_copy(0, 0).start()
    idx_copy(1, 1).start()
    idx_copy(0, 0).wait()
    gather_copy(0).start()

    def pair_body(i, carry):
        for b in range(2):
            t = 2 * i + b
            gather_copy(b).wait()

            @pl.when(t + 1 < N_CHUNKS)
            def _():
                idx_copy(t + 1, 1 - b).wait()

                @pl.when(t >= 1)
                def _():
                    out_copy(t - 1, 1 - b).wait()

                gather_copy(1 - b).start()

            compute(b)

            @pl.when(t + 2 < N_CHUNKS)
            def _():
                idx_copy(t + 2, b).start()

            out_copy(t, b).start()
        return carry

    lax.fori_loop(0, N_CHUNKS // 2, pair_body, 0)
    out_copy(N_CHUNKS - 2, 0).wait()
    out_copy(N_CHUNKS - 1, 1).wait()


@jax.jit
def _emb(x_flat, E):
    mesh = plsc.VectorSubcoreMesh(core_axis_name="c", subcore_axis_name="s")
    f = functools.partial(
        pl.kernel,
        mesh=mesh,
        compiler_params=pltpu.CompilerParams(
            needs_layout_passes=False, use_tc_tiling_on_sc=False),
        out_type=jax.ShapeDtypeStruct((B, 2 * H), jnp.float32),
        scratch_types=[
            pltpu.VMEM((CHUNK,), jnp.int32),
            pltpu.VMEM((CHUNK,), jnp.int32),
            pltpu.VMEM((CHUNK, H), jnp.float32),
            pltpu.VMEM((CHUNK, H), jnp.float32),
            pltpu.SemaphoreType.DMA,
            pltpu.SemaphoreType.DMA,
            pltpu.SemaphoreType.DMA,
            pltpu.SemaphoreType.DMA,
            pltpu.SemaphoreType.DMA,
            pltpu.SemaphoreType.DMA,
        ],
    )(_emb_body)
    return f(x_flat, E)


def kernel(x, E):
    NS_PREP = 4
    S = VOCAB // NS_PREP
    halves = [
        jnp.pad(E[i * S:(i + 1) * S], ((0, 0), (0, H))) * SCALE
        for i in range(NS_PREP)
    ]
    Ep = jnp.concatenate(halves, axis=0).reshape(2 * VOCAB, H)
    y = _emb(x.reshape(-1) * 2, Ep)
    return y.reshape(x.shape + (2 * H,))[..., :H]


# speedup vs baseline: 2.8860x; 2.8860x over previous
import functools

import jax
import jax.numpy as jnp
from jax import lax
from jax.experimental import pallas as pl
from jax.experimental.pallas import tpu as pltpu
from jax.experimental.pallas import tpu_sc as plsc

VOCAB = 1000000
H = 64
PAD = 0
SCALE = float(H) ** 0.5

B = 4096 * 200
NC, NS, L = 2, 16, 16
NW = NC * NS
B_PER_W = B // NW
CHUNK = 800
N_CHUNKS = B_PER_W // CHUNK
GRP = CHUNK // L


def _emb_body(idx_hbm, table_hbm, out_hbm,
              idx0, idx1, rows0, rows1,
              s_i0, s_i1, s_g0, s_g1, s_o0, s_o1):
    wid = lax.axis_index("c") * NS + lax.axis_index("s")
    base0 = wid * B_PER_W
    idx_b, rows_b = (idx0, idx1), (rows0, rows1)
    s_i, s_g, s_o = (s_i0, s_i1), (s_g0, s_g1), (s_o0, s_o1)

    def idx_copy(t, b):
        return pltpu.make_async_copy(
            idx_hbm.at[pl.ds(base0 + t * CHUNK, CHUNK)], idx_b[b], s_i[b])

    def gather_copy(b):
        return pltpu.make_async_copy(table_hbm.at[idx_b[b]], rows_b[b], s_g[b])

    def out_copy(t, b):
        return pltpu.make_async_copy(
            rows_b[b],
            out_hbm.at[pl.ds(base0 + t * CHUNK, CHUNK), pl.ds(0, H)],
            s_o[b])

    def compute(b):
        idx_v, rows_v = idx_b[b], rows_b[b]

        def grp_body(g, c2):
            row0 = g * L
            idxv = idx_v[pl.ds(row0, L)]
            pad_m = idxv == PAD
            npad = jnp.sum(jnp.where(pad_m, 1, 0).astype(jnp.int32))

            @pl.when(npad > 0)
            def _():
                rowids = row0 + lax.iota(jnp.int32, L)
                zeros = jnp.zeros((L,), jnp.float32)
                for c in range(H):
                    colids = jnp.full((L,), c, jnp.int32)
                    plsc.store_scatter(rows_v, [rowids, colids], zeros,
                                       mask=pad_m)
            return c2

        lax.fori_loop(0, GRP, grp_body, 0)

    idx_copy(0, 0).start()
    idx_copy(1, 1).start()
    idx_copy(0, 0).wait()
    gather_copy(0).start()

    def pair_body(i, carry):
        for b in range(2):
            t = 2 * i + b
            gather_copy(b).wait()

            @pl.when(t + 1 < N_CHUNKS)
            def _():
                idx_copy(t + 1, 1 - b).wait()

                @pl.when(t >= 1)
                def _():
                    out_copy(t - 1, 1 - b).wait()

                gather_copy(1 - b).start()

            compute(b)

            @pl.when(t + 2 < N_CHUNKS)
            def _():
                idx_copy(t + 2, b).start()

            out_copy(t, b).start()
        return carry

    lax.fori_loop(0, N_CHUNKS // 2, pair_body, 0)
    out_copy(N_CHUNKS - 2, 0).wait()
    out_copy(N_CHUNKS - 1, 1).wait()


@jax.jit
def _emb(x_flat, E):
    mesh = plsc.VectorSubcoreMesh(core_axis_name="c", subcore_axis_name="s")
    f = functools.partial(
        pl.kernel,
        mesh=mesh,
        compiler_params=pltpu.CompilerParams(
            needs_layout_passes=False, use_tc_tiling_on_sc=False),
        out_type=jax.ShapeDtypeStruct((B, 2 * H), jnp.float32),
        scratch_types=[
            pltpu.VMEM((CHUNK,), jnp.int32),
            pltpu.VMEM((CHUNK,), jnp.int32),
            pltpu.VMEM((CHUNK, H), jnp.float32),
            pltpu.VMEM((CHUNK, H), jnp.float32),
            pltpu.SemaphoreType.DMA,
            pltpu.SemaphoreType.DMA,
            pltpu.SemaphoreType.DMA,
            pltpu.SemaphoreType.DMA,
            pltpu.SemaphoreType.DMA,
            pltpu.SemaphoreType.DMA,
        ],
    )(_emb_body)
    return f(x_flat, E)


def kernel(x, E):
    Ep = (jnp.pad(E, ((0, 0), (0, H))) * SCALE).reshape(2 * VOCAB, H)
    y = _emb(x.reshape(-1) * 2, Ep)
    return y.reshape(x.shape + (2 * H,))[..., :H]


# speedup vs baseline: 3.6596x; 1.2681x over previous
import functools

import jax
import jax.numpy as jnp
from jax import lax
from jax.experimental import pallas as pl
from jax.experimental.pallas import tpu as pltpu
from jax.experimental.pallas import tpu_sc as plsc

VOCAB = 1000000
H = 64
PAD = 0
SCALE = float(H) ** 0.5

B = 4096 * 200
NC, NS, L = 2, 16, 16
NW = NC * NS
B_PER_W = B // NW
CHUNK = 800
N_CHUNKS = B_PER_W // CHUNK
GRP = CHUNK // L


def _emb_body(idx_hbm, table_hbm, out_hbm,
              idx0, idx1, rows0, rows1,
              s_i0, s_i1, s_g0, s_g1, s_o0, s_o1):
    wid = lax.axis_index("c") * NS + lax.axis_index("s")
    base0 = wid * B_PER_W
    idx_b, rows_b = (idx0, idx1), (rows0, rows1)
    s_i, s_g, s_o = (s_i0, s_i1), (s_g0, s_g1), (s_o0, s_o1)

    def idx_copy(t, b):
        return pltpu.make_async_copy(
            idx_hbm.at[pl.ds(base0 + t * CHUNK, CHUNK)], idx_b[b], s_i[b])

    def gather_copy(b):
        return pltpu.make_async_copy(table_hbm.at[idx_b[b]], rows_b[b], s_g[b])

    def out_copy(t, b):
        return pltpu.make_async_copy(
            rows_b[b],
            out_hbm.at[pl.ds(base0 + t * CHUNK, CHUNK), pl.ds(0, H)],
            s_o[b])

    def compute(b):
        idx_v, rows_v = idx_b[b], rows_b[b]

        def grp_body(g, c2):
            row0 = g * L
            idxv = idx_v[pl.ds(row0, L)]
            pad_m = idxv == PAD
            npad = jnp.sum(jnp.where(pad_m, 1, 0).astype(jnp.int32))

            @pl.when(npad > 0)
            def _():
                rowids = row0 + lax.iota(jnp.int32, L)
                zeros = jnp.zeros((L,), jnp.float32)
                for c in range(H):
                    colids = jnp.full((L,), c, jnp.int32)
                    plsc.store_scatter(rows_v, [rowids, colids], zeros,
                                       mask=pad_m)
            return c2

        lax.fori_loop(0, GRP, grp_body, 0)

    idx_copy(0, 0).start()
    idx_copy(1, 1).start()
    idx_copy(0, 0).wait()
    gather_copy(0).start()

    def pair_body(i, carry):
        for b in range(2):
            t = 2 * i + b
            gather_copy(b).wait()

            @pl.when(t + 1 < N_CHUNKS)
            def _():
                idx_copy(t + 1, 1 - b).wait()

                @pl.when(t >= 1)
                def _():
                    out_copy(t - 1, 1 - b).wait()

                gather_copy(1 - b).start()

            compute(b)

            @pl.when(t + 2 < N_CHUNKS)
            def _():
                idx_copy(t + 2, b).start()

            out_copy(t, b).start()
        return carry

    lax.fori_loop(0, N_CHUNKS // 2, pair_body, 0)
    out_copy(N_CHUNKS - 2, 0).wait()
    out_copy(N_CHUNKS - 1, 1).wait()


@jax.jit
def _emb(x_flat, E):
    mesh = plsc.VectorSubcoreMesh(core_axis_name="c", subcore_axis_name="s")
    f = functools.partial(
        pl.kernel,
        mesh=mesh,
        compiler_params=pltpu.CompilerParams(
            needs_layout_passes=False, use_tc_tiling_on_sc=False),
        out_type=jax.ShapeDtypeStruct((B, 2 * H), jnp.float32),
        scratch_types=[
            pltpu.VMEM((CHUNK,), jnp.int32),
            pltpu.VMEM((CHUNK,), jnp.int32),
            pltpu.VMEM((CHUNK, H), jnp.float32),
            pltpu.VMEM((CHUNK, H), jnp.float32),
            pltpu.SemaphoreType.DMA,
            pltpu.SemaphoreType.DMA,
            pltpu.SemaphoreType.DMA,
            pltpu.SemaphoreType.DMA,
            pltpu.SemaphoreType.DMA,
            pltpu.SemaphoreType.DMA,
        ],
    )(_emb_body)
    return f(x_flat, E)


FMT_BLK = 4096


def _fmt_body(et_ref, out_ref):
    blk = et_ref[...]
    out_ref[:, :H] = jnp.transpose(blk) * SCALE
    out_ref[:, H:] = jnp.zeros((FMT_BLK, H), jnp.float32)


@jax.jit
def _fmt(Et):
    grid = (VOCAB + FMT_BLK - 1) // FMT_BLK
    return pl.pallas_call(
        _fmt_body,
        grid=(grid,),
        in_specs=[pl.BlockSpec((H, FMT_BLK), lambda i: (0, i))],
        out_specs=pl.BlockSpec((FMT_BLK, 2 * H), lambda i: (i, 0)),
        out_shape=jax.ShapeDtypeStruct((VOCAB, 2 * H), jnp.float32),
    )(Et)


def kernel(x, E):
    Ep = _fmt(E.T).reshape(2 * VOCAB, H)
    y = _emb(x.reshape(-1) * 2, Ep)
    return y.reshape(x.shape + (2 * H,))[..., :H]
